# Initial kernel scaffold; baseline (speedup 1.0000x reference)
import jax
import jax.numpy as jnp
from jax.experimental import pallas as pl


def kernel(z, sd_coupling, d_filling_n, e_conductivity_n, d_filling_mult, z_embed_weight):
    raise NotImplementedError("write your pallas kernel here")


# speedup vs baseline: 1.8923x; 1.8923x over previous
import jax
import jax.numpy as jnp
from jax import lax
from jax.experimental import pallas as pl
from jax.experimental.pallas import tpu as pltpu
from jax.experimental.pallas import tpu_sc as plsc

N = 100000
VOCAB = 100
D = 16
OUT_D = 20

_info = plsc.get_sparse_core_info()
_NC, _NS, _L = _info.num_cores, _info.num_subcores, _info.num_lanes
_NW = _NC * _NS
FULL = 3200
LAST = N - (_NW - 1) * FULL


def _sc_body(z_hbm, sd_hbm, df_hbm, cond_hbm, mult_hbm, tab_hbm, out_hbm,
             tab_v, idx_v, sd_v, df_v, cond_v, mult_v, out_v):
    wid = lax.axis_index("s") * _NC + lax.axis_index("c")
    base = wid * FULL
    is_last = wid == _NW - 1

    pltpu.sync_copy(tab_hbm, tab_v)

    @pl.when(jnp.logical_not(is_last))
    def _():
        pltpu.sync_copy(z_hbm.at[pl.ds(base, FULL)], idx_v)
        pltpu.sync_copy(sd_hbm.at[pl.ds(base, FULL)], sd_v)
        pltpu.sync_copy(df_hbm.at[pl.ds(base, FULL)], df_v)
        pltpu.sync_copy(cond_hbm.at[pl.ds(base, FULL)], cond_v)
        pltpu.sync_copy(mult_hbm.at[pl.ds(base, FULL)], mult_v)

    @pl.when(is_last)
    def _():
        pltpu.sync_copy(z_hbm.at[pl.ds(base, LAST)], idx_v.at[pl.ds(0, LAST)])
        pltpu.sync_copy(sd_hbm.at[pl.ds(base, LAST)], sd_v.at[pl.ds(0, LAST)])
        pltpu.sync_copy(df_hbm.at[pl.ds(base, LAST)], df_v.at[pl.ds(0, LAST)])
        pltpu.sync_copy(cond_hbm.at[pl.ds(base, LAST)], cond_v.at[pl.ds(0, LAST)])
        pltpu.sync_copy(mult_hbm.at[pl.ds(base, LAST)], mult_v.at[pl.ds(0, LAST)])

    iota20 = lax.iota(jnp.int32, _L) * OUT_D
    nblk = jnp.where(is_last, LAST // _L, FULL // _L)

    def blk(k, carry):
        idx16 = idx_v[pl.ds(k * _L, _L)]
        addr = idx16 * D
        offb = k * (_L * OUT_D) + iota20
        vals = [plsc.load_gather(tab_v, [addr + g]) for g in range(D)]
        cols = [cv[pl.ds(k * _L, _L)] for cv in (sd_v, df_v, cond_v, mult_v)]
        for g in range(D):
            plsc.store_scatter(out_v, [offb + g], vals[g])
        for c in range(4):
            plsc.store_scatter(out_v, [offb + D + c], cols[c])
        return carry

    lax.fori_loop(0, nblk, blk, 0)

    @pl.when(jnp.logical_not(is_last))
    def _():
        pltpu.sync_copy(out_v, out_hbm.at[pl.ds(base * OUT_D, FULL * OUT_D)])

    @pl.when(is_last)
    def _():
        pltpu.sync_copy(out_v.at[pl.ds(0, LAST * OUT_D)],
                        out_hbm.at[pl.ds(base * OUT_D, LAST * OUT_D)])


_sc_call = pl.kernel(
    _sc_body,
    mesh=plsc.VectorSubcoreMesh(core_axis_name="c", subcore_axis_name="s"),
    compiler_params=pltpu.CompilerParams(needs_layout_passes=False),
    out_type=jax.ShapeDtypeStruct((N * OUT_D,), jnp.float32),
    scratch_types=[
        pltpu.VMEM((VOCAB * D,), jnp.float32),
        pltpu.VMEM((FULL,), jnp.int32),
        pltpu.VMEM((FULL,), jnp.float32),
        pltpu.VMEM((FULL,), jnp.float32),
        pltpu.VMEM((FULL,), jnp.float32),
        pltpu.VMEM((FULL,), jnp.float32),
        pltpu.VMEM((FULL * OUT_D,), jnp.float32),
    ],
)


def kernel(z, sd_coupling, d_filling_n, e_conductivity_n, d_filling_mult,
           z_embed_weight):
    out = _sc_call(
        z.astype(jnp.int32).reshape(N),
        sd_coupling.reshape(N),
        d_filling_n.reshape(N),
        e_conductivity_n.reshape(N),
        d_filling_mult.reshape(N),
        z_embed_weight.reshape(VOCAB * D),
    )
    return out.reshape(N, OUT_D)


# speedup vs baseline: 5.0691x; 2.6789x over previous
import jax
import jax.numpy as jnp
from jax import lax
from jax.experimental import pallas as pl
from jax.experimental.pallas import tpu as pltpu
from jax.experimental.pallas import tpu_sc as plsc

N = 100000
VOCAB = 100
D = 16
OUT_D = 20

_info = plsc.get_sparse_core_info()
_NC, _NS, _L = _info.num_cores, _info.num_subcores, _info.num_lanes
_NW = _NC * _NS
FULL = 3200
LAST = N - (_NW - 1) * FULL


def _sc_body(z_hbm, sd_hbm, df_hbm, cond_hbm, mult_hbm, tab_hbm, out_hbm,
             tab_v, idx_v, out_v, sem):
    wid = lax.axis_index("s") * _NC + lax.axis_index("c")
    base = wid * FULL
    is_last = wid == _NW - 1

    pltpu.sync_copy(tab_hbm, tab_v)

    @pl.when(jnp.logical_not(is_last))
    def _():
        cps = [pltpu.async_copy(z_hbm.at[pl.ds(base, FULL)], idx_v, sem)]
        for c, col in enumerate((sd_hbm, df_hbm, cond_hbm, mult_hbm)):
            cps.append(pltpu.async_copy(
                col.at[pl.ds(base, FULL)],
                out_v.at[pl.ds((D + c) * FULL, FULL)], sem))
        for cp in cps:
            cp.wait()

    @pl.when(is_last)
    def _():
        cps = [pltpu.async_copy(z_hbm.at[pl.ds(base, LAST)],
                                idx_v.at[pl.ds(0, LAST)], sem)]
        for c, col in enumerate((sd_hbm, df_hbm, cond_hbm, mult_hbm)):
            cps.append(pltpu.async_copy(
                col.at[pl.ds(base, LAST)],
                out_v.at[pl.ds((D + c) * FULL, LAST)], sem))
        for cp in cps:
            cp.wait()

    nblk = jnp.where(is_last, LAST // _L, FULL // _L)

    def blk(k, carry):
        idx16 = idx_v[pl.ds(k * _L, _L)]
        addr = idx16 * D
        vals = [plsc.load_gather(tab_v, [addr + g]) for g in range(D)]
        for g in range(D):
            out_v[pl.ds(g * FULL + k * _L, _L)] = vals[g]
        return carry

    lax.fori_loop(0, nblk, blk, 0)

    @pl.when(jnp.logical_not(is_last))
    def _():
        cps = [pltpu.async_copy(out_v.at[pl.ds(g * FULL, FULL)],
                                out_hbm.at[pl.ds(g * N + base, FULL)], sem)
               for g in range(OUT_D)]
        for cp in cps:
            cp.wait()

    @pl.when(is_last)
    def _():
        cps = [pltpu.async_copy(out_v.at[pl.ds(g * FULL, LAST)],
                                out_hbm.at[pl.ds(g * N + base, LAST)], sem)
               for g in range(OUT_D)]
        for cp in cps:
            cp.wait()


_sc_call = pl.kernel(
    _sc_body,
    mesh=plsc.VectorSubcoreMesh(core_axis_name="c", subcore_axis_name="s"),
    compiler_params=pltpu.CompilerParams(needs_layout_passes=False),
    out_type=jax.ShapeDtypeStruct((OUT_D * N,), jnp.float32),
    scratch_types=[
        pltpu.VMEM((VOCAB * D,), jnp.float32),
        pltpu.VMEM((FULL,), jnp.int32),
        pltpu.VMEM((OUT_D * FULL,), jnp.float32),
        pltpu.SemaphoreType.DMA,
    ],
)


def kernel(z, sd_coupling, d_filling_n, e_conductivity_n, d_filling_mult,
           z_embed_weight):
    out = _sc_call(
        z.astype(jnp.int32),
        sd_coupling.reshape(N),
        d_filling_n.reshape(N),
        e_conductivity_n.reshape(N),
        d_filling_mult.reshape(N),
        z_embed_weight.reshape(VOCAB * D),
    )
    return out.reshape(OUT_D, N).T


# speedup vs baseline: 5.1557x; 1.0171x over previous
import jax
import jax.numpy as jnp
from jax import lax
from jax.experimental import pallas as pl
from jax.experimental.pallas import tpu as pltpu
from jax.experimental.pallas import tpu_sc as plsc

N = 100000
VOCAB = 100
D = 16
OUT_D = 20

_info = plsc.get_sparse_core_info()
_NC, _NS, _L = _info.num_cores, _info.num_subcores, _info.num_lanes
_NW = _NC * _NS
FULL = 3200
LAST = N - (_NW - 1) * FULL


def _sc_body(z_hbm, sd_hbm, df_hbm, cond_hbm, mult_hbm, tab_hbm, out_hbm,
             tab_v, idx_v, out_v, sem_in, sem_cols, sem_out):
    wid = lax.axis_index("s") * _NC + lax.axis_index("c")
    base = wid * FULL
    is_last = wid == _NW - 1

    def run(rows):
        in_cps = [
            pltpu.async_copy(tab_hbm, tab_v, sem_in),
            pltpu.async_copy(z_hbm.at[pl.ds(base, rows)],
                             idx_v.at[pl.ds(0, rows)], sem_in),
        ]
        col_cps = [
            pltpu.async_copy(col.at[pl.ds(base, rows)],
                             out_v.at[pl.ds((D + c) * FULL, rows)], sem_cols)
            for c, col in enumerate((sd_hbm, df_hbm, cond_hbm, mult_hbm))
        ]
        for cp in in_cps:
            cp.wait()

        half = rows // 2
        nblk_h = half // _L

        def blk(k, carry):
            idx16 = idx_v[pl.ds(k * _L, _L)]
            addr = idx16 * D
            vals = [plsc.load_gather(tab_v, [addr + g]) for g in range(D)]
            for g in range(D):
                out_v[pl.ds(g * FULL + k * _L, _L)] = vals[g]
            return carry

        lax.fori_loop(0, nblk_h, blk, 0)

        cps = [pltpu.async_copy(out_v.at[pl.ds(g * FULL, half)],
                                out_hbm.at[pl.ds(g * N + base, half)],
                                sem_out)
               for g in range(D)]

        lax.fori_loop(nblk_h, 2 * nblk_h, blk, 0)

        for cp in col_cps:
            cp.wait()
        cps += [pltpu.async_copy(out_v.at[pl.ds(g * FULL + half, half)],
                                 out_hbm.at[pl.ds(g * N + base + half, half)],
                                 sem_out)
                for g in range(D)]
        cps += [pltpu.async_copy(out_v.at[pl.ds(g * FULL, rows)],
                                 out_hbm.at[pl.ds(g * N + base, rows)],
                                 sem_out)
                for g in range(D, OUT_D)]
        for cp in cps:
            cp.wait()

    @pl.when(jnp.logical_not(is_last))
    def _():
        run(FULL)

    @pl.when(is_last)
    def _():
        run(LAST)


_sc_call = pl.kernel(
    _sc_body,
    mesh=plsc.VectorSubcoreMesh(core_axis_name="c", subcore_axis_name="s"),
    compiler_params=pltpu.CompilerParams(needs_layout_passes=False),
    out_type=jax.ShapeDtypeStruct((OUT_D * N,), jnp.float32),
    scratch_types=[
        pltpu.VMEM((VOCAB * D,), jnp.float32),
        pltpu.VMEM((FULL,), jnp.int32),
        pltpu.VMEM((OUT_D * FULL,), jnp.float32),
        pltpu.SemaphoreType.DMA,
        pltpu.SemaphoreType.DMA,
        pltpu.SemaphoreType.DMA,
    ],
)


def kernel(z, sd_coupling, d_filling_n, e_conductivity_n, d_filling_mult,
           z_embed_weight):
    out = _sc_call(
        z.astype(jnp.int32),
        sd_coupling.reshape(N),
        d_filling_n.reshape(N),
        e_conductivity_n.reshape(N),
        d_filling_mult.reshape(N),
        z_embed_weight.reshape(VOCAB * D),
    )
    return out.reshape(OUT_D, N).T


# speedup vs baseline: 5.3757x; 1.0427x over previous
import jax
import jax.numpy as jnp
from jax import lax
from jax.experimental import pallas as pl
from jax.experimental.pallas import tpu as pltpu
from jax.experimental.pallas import tpu_sc as plsc

N = 100000
VOCAB = 100
D = 16
OUT_D = 20

_info = plsc.get_sparse_core_info()
_NC, _NS, _L = _info.num_cores, _info.num_subcores, _info.num_lanes
_NW = _NC * _NS
FULL = 3200
LAST = N - (_NW - 1) * FULL


def _sc_body(z_hbm, sd_hbm, df_hbm, cond_hbm, mult_hbm, tab_hbm, out_hbm,
             tab_v, idx_v, out_v, sem_in, sem_cols, sem_out):
    wid = lax.axis_index("s") * _NC + lax.axis_index("c")
    base = wid * FULL
    is_last = wid == _NW - 1

    def run(rows):
        in_cps = [
            pltpu.async_copy(tab_hbm, tab_v, sem_in),
            pltpu.async_copy(z_hbm.at[pl.ds(base, rows)],
                             idx_v.at[pl.ds(0, rows)], sem_in),
        ]
        col_cps = [
            pltpu.async_copy(col.at[pl.ds(base, rows)],
                             out_v.at[pl.ds((D + c) * FULL, rows)], sem_cols)
            for c, col in enumerate((sd_hbm, df_hbm, cond_hbm, mult_hbm))
        ]
        for cp in in_cps:
            cp.wait()

        half = rows // 2
        nblk_h = half // _L

        def gather_blocks(lo, hi):
            @plsc.parallel_loop(lo, hi, unroll=2)
            def _body(k):
                idx16 = idx_v[pl.ds(k * _L, _L)]
                addr = idx16 * D
                vals = [plsc.load_gather(tab_v, [addr + g]) for g in range(D)]
                for g in range(D):
                    out_v[pl.ds(g * FULL + k * _L, _L)] = vals[g]

        gather_blocks(0, nblk_h)

        cps = [pltpu.async_copy(out_v.at[pl.ds(g * FULL, half)],
                                out_hbm.at[pl.ds(g * N + base, half)],
                                sem_out)
               for g in range(D)]

        gather_blocks(nblk_h, 2 * nblk_h)

        for cp in col_cps:
            cp.wait()
        cps += [pltpu.async_copy(out_v.at[pl.ds(g * FULL + half, half)],
                                 out_hbm.at[pl.ds(g * N + base + half, half)],
                                 sem_out)
                for g in range(D)]
        cps += [pltpu.async_copy(out_v.at[pl.ds(g * FULL, rows)],
                                 out_hbm.at[pl.ds(g * N + base, rows)],
                                 sem_out)
                for g in range(D, OUT_D)]
        for cp in cps:
            cp.wait()

    @pl.when(jnp.logical_not(is_last))
    def _():
        run(FULL)

    @pl.when(is_last)
    def _():
        run(LAST)


_sc_call = pl.kernel(
    _sc_body,
    mesh=plsc.VectorSubcoreMesh(core_axis_name="c", subcore_axis_name="s"),
    compiler_params=pltpu.CompilerParams(needs_layout_passes=False),
    out_type=jax.ShapeDtypeStruct((OUT_D * N,), jnp.float32),
    scratch_types=[
        pltpu.VMEM((VOCAB * D,), jnp.float32),
        pltpu.VMEM((FULL,), jnp.int32),
        pltpu.VMEM((OUT_D * FULL,), jnp.float32),
        pltpu.SemaphoreType.DMA,
        pltpu.SemaphoreType.DMA,
        pltpu.SemaphoreType.DMA,
    ],
)


def kernel(z, sd_coupling, d_filling_n, e_conductivity_n, d_filling_mult,
           z_embed_weight):
    out = _sc_call(
        z.astype(jnp.int32),
        sd_coupling.reshape(N),
        d_filling_n.reshape(N),
        e_conductivity_n.reshape(N),
        d_filling_mult.reshape(N),
        z_embed_weight.reshape(VOCAB * D),
    )
    return out.reshape(OUT_D, N).T
